# Initial kernel scaffold; baseline (speedup 1.0000x reference)
import functools

import jax
import jax.numpy as jnp
from jax import lax
from jax.experimental import pallas as pl
from jax.experimental.pallas import tpu as pltpu
from jax.experimental.pallas import tpu_sc as plsc

_E = 320000
_D = 128
_G = 64
_NTILES = 16
_M = _E // _NTILES

_BE = 8000
_BP = 8000


def _logits_body(x_ref, w_ref, b_ref, o_ref):
    x = x_ref[...]
    z = jnp.dot(x, w_ref[...], preferred_element_type=jnp.float32)
    z = z + b_ref[0, 0]
    o_ref[...] = jnp.where(z >= 0, z, 0.01 * z)


def _logits(edge_feats, W, b):
    return pl.pallas_call(
        _logits_body,
        grid=(_E // _BE,),
        in_specs=[
            pl.BlockSpec((_BE, _D), lambda i: (i, 0)),
            pl.BlockSpec((_D, 1), lambda i: (0, 0)),
            pl.BlockSpec((1, 1), lambda i: (0, 0)),
        ],
        out_specs=pl.BlockSpec((_BE, 1), lambda i: (i, 0)),
        out_shape=jax.ShapeDtypeStruct((_E, 1), jnp.float32),
    )(edge_feats, W, b.reshape(1, 1))


def _take16(x, idx):
    return lax.gather(
        x,
        idx[:, None],
        lax.GatherDimensionNumbers(
            offset_dims=(), collapsed_slice_dims=(0,), start_index_map=(0,)
        ),
        (1,),
        mode=lax.GatherScatterMode.PROMISE_IN_BOUNDS,
    )


def _sc_softmax_body(l_hbm, dst_hbm, w_hbm, dstb, lb, exb, ridb, cb, totb,
                     pubb, tblb, shtbl):
    wid = lax.axis_index("s")
    start = wid * _M
    lane = lax.iota(jnp.int32, 16)
    zeros16 = jnp.zeros((16,), jnp.int32)
    idx15 = jnp.full((16,), 15, jnp.int32)

    pltpu.sync_copy(dst_hbm.at[pl.ds(start, _M)], dstb.at[pl.ds(0, _M)])
    pltpu.sync_copy(l_hbm.at[pl.ds(start, _M)], lb)
    dstb[pl.ds(_M, 16)] = jnp.full((16,), -1, jnp.int32)
    cb[pl.ds(0, 16)] = jnp.zeros((16,), jnp.float32)

    def step_a(v, carry):
        csum, rbase = carry
        base = v * 16
        dv = dstb[pl.ds(base, 16)]
        dn = dstb[pl.ds(base + 1, 16)]
        ex = jnp.exp(lb[pl.ds(base, 16)])
        exb[pl.ds(base, 16)] = ex
        lastm = dv != dn
        lasti = lastm.astype(jnp.int32)
        c = plsc.cumsum(ex) + csum
        pcnt = plsc.cumsum(lasti)
        rid = rbase + pcnt - lasti
        ridb[pl.ds(base, 16)] = rid
        plsc.store_scatter(cb, [rid + 1], c, mask=lastm)
        return _take16(c, idx15), rbase + _take16(pcnt, idx15)

    csum0 = jnp.zeros((16,), jnp.float32)
    rb0 = jnp.zeros((16,), jnp.int32)
    _, rbF = lax.fori_loop(0, _M // 16, step_a, (csum0, rb0))
    num_runs = jnp.max(rbF)

    def step_b(r, _):
        base = r * 16
        hi = cb[pl.ds(base + 1, 16)]
        lo = cb[pl.ds(base, 16)]
        totb[pl.ds(base, 16)] = hi - lo
        return 0

    lax.fori_loop(0, (num_runs + 15) // 16, step_b, 0)

    d_first = _take16(dstb[pl.ds(0, 16)], zeros16)
    d_last = _take16(dstb[pl.ds(_M - 16, 16)], idx15)
    f_tot = plsc.load_gather(totb, [zeros16])
    last_idx = jnp.full((16,), num_runs - 1, jnp.int32)
    l_tot = plsc.load_gather(totb, [last_idx])
    l_pub = jnp.where(num_runs > 1, l_tot, jnp.zeros((16,), jnp.float32))
    pub = jnp.where(lane == 0, plsc.bitcast(d_first, jnp.float32),
          jnp.where(lane == 1, f_tot,
          jnp.where(lane == 2, plsc.bitcast(d_last, jnp.float32),
          jnp.where(lane == 3, l_pub, jnp.zeros((16,), jnp.float32)))))
    pubb[...] = pub
    pltpu.sync_copy(pubb.at[pl.ds(0, 8)], shtbl.at[wid])
    plsc.subcore_barrier()
    pltpu.sync_copy(shtbl, tblb)

    fdst = plsc.bitcast(plsc.load_gather(tblb, [lane, zeros16]), jnp.int32)
    fsum = plsc.load_gather(tblb, [lane, zeros16 + 1])
    ldst = plsc.bitcast(plsc.load_gather(tblb, [lane, zeros16 + 2]), jnp.int32)
    lsum = plsc.load_gather(tblb, [lane, zeros16 + 3])

    def seg_total(d):
        zf = jnp.zeros((16,), jnp.float32)
        contrib = jnp.where(fdst == d, fsum, zf) + jnp.where(ldst == d, lsum, zf)
        return jnp.sum(contrib)

    cf = seg_total(d_first)
    cl = seg_total(d_last)
    lane0 = lane == 0
    plsc.store_scatter(totb, [zeros16], jnp.full((16,), cf), mask=lane0)
    plsc.store_scatter(totb, [last_idx], jnp.full((16,), cl), mask=lane0)

    def step_c(v, _):
        base = v * 16
        rid = ridb[pl.ds(base, 16)]
        tot = plsc.load_gather(totb, [rid])
        lb[pl.ds(base, 16)] = exb[pl.ds(base, 16)] / tot
        return 0

    lax.fori_loop(0, _M // 16, step_c, 0)
    pltpu.sync_copy(lb, w_hbm.at[pl.ds(start, _M)])


def _sc_softmax(l_flat, dst):
    mesh = plsc.VectorSubcoreMesh(
        core_axis_name="c", subcore_axis_name="s", num_cores=1
    )
    fn = functools.partial(
        pl.kernel,
        mesh=mesh,
        out_type=jax.ShapeDtypeStruct((_E,), jnp.float32),
        scratch_types=[
            pltpu.VMEM((_M + 16,), jnp.int32),
            pltpu.VMEM((_M,), jnp.float32),
            pltpu.VMEM((_M,), jnp.float32),
            pltpu.VMEM((_M,), jnp.int32),
            pltpu.VMEM((_M + 16,), jnp.float32),
            pltpu.VMEM((_M + 16,), jnp.float32),
            pltpu.VMEM((16,), jnp.float32),
            pltpu.VMEM((16, 8), jnp.float32),
            pltpu.VMEM_SHARED((16, 8), jnp.float32),
        ],
    )(_sc_softmax_body)
    return fn(l_flat, dst)


def _pool_body(g_ref, w_ref, x_ref, o_ref):
    @pl.when(pl.program_id(0) == 0)
    def _init():
        o_ref[...] = jnp.zeros_like(o_ref)

    g = g_ref[...]
    onehot = (g == lax.broadcasted_iota(jnp.int32, (_BP, _G), 1)).astype(
        jnp.float32
    )
    wx = x_ref[...] * w_ref[...]
    o_ref[...] += lax.dot_general(
        onehot, wx, (((0,), (0,)), ((), ())), preferred_element_type=jnp.float32
    )


def _pool(graph_ids2d, w2d, edge_feats):
    return pl.pallas_call(
        _pool_body,
        grid=(_E // _BP,),
        in_specs=[
            pl.BlockSpec((_BP, 1), lambda i: (i, 0)),
            pl.BlockSpec((_BP, 1), lambda i: (i, 0)),
            pl.BlockSpec((_BP, _D), lambda i: (i, 0)),
        ],
        out_specs=pl.BlockSpec((_G, _D), lambda i: (0, 0)),
        out_shape=jax.ShapeDtypeStruct((_G, _D), jnp.float32),
    )(graph_ids2d, w2d, edge_feats)


def kernel(edge_feats, dst, graph_ids, W, b):
    dst = dst.astype(jnp.int32)
    graph_ids = graph_ids.astype(jnp.int32)
    logits = _logits(edge_feats, W, b)
    w_flat = _sc_softmax(logits.reshape(_E), dst)
    w2d = w_flat.reshape(_E, 1)
    h = _pool(graph_ids.reshape(_E, 1), w2d, edge_feats)
    return (h, w2d)


# speedup vs baseline: 9.0485x; 9.0485x over previous
import functools

import jax
import jax.numpy as jnp
from jax import lax
from jax.experimental import pallas as pl
from jax.experimental.pallas import tpu as pltpu
from jax.experimental.pallas import tpu_sc as plsc

_E = 320000
_D = 128
_G = 64
_NTILES = 16
_M = _E // _NTILES

_BE = 8000
_BP = 8000


def _logits_body(x_ref, w_ref, b_ref, o_ref):
    x = x_ref[...]
    z = jnp.dot(x, w_ref[...], preferred_element_type=jnp.float32)
    z = z + b_ref[0, 0]
    o_ref[...] = jnp.where(z >= 0, z, 0.01 * z)


def _logits(edge_feats, W, b):
    return pl.pallas_call(
        _logits_body,
        grid=(_E // _BE,),
        in_specs=[
            pl.BlockSpec((_BE, _D), lambda i: (i, 0)),
            pl.BlockSpec((_D, 1), lambda i: (0, 0)),
            pl.BlockSpec((1, 1), lambda i: (0, 0)),
        ],
        out_specs=pl.BlockSpec((_BE, 1), lambda i: (i, 0)),
        out_shape=jax.ShapeDtypeStruct((_E, 1), jnp.float32),
    )(edge_feats, W, b.reshape(1, 1))


def _take16(x, idx):
    return lax.gather(
        x,
        idx[:, None],
        lax.GatherDimensionNumbers(
            offset_dims=(), collapsed_slice_dims=(0,), start_index_map=(0,)
        ),
        (1,),
        mode=lax.GatherScatterMode.PROMISE_IN_BOUNDS,
    )


def _sc_softmax_body(l_hbm, dst_hbm, w_hbm, dstb, lb, exb, ridb, cb, totb,
                     pubb, tblb, shtbl):
    wid = lax.axis_index("s")
    start = wid * _M
    lane = lax.iota(jnp.int32, 16)
    zeros16 = jnp.zeros((16,), jnp.int32)
    idx15 = jnp.full((16,), 15, jnp.int32)

    pltpu.sync_copy(dst_hbm.at[pl.ds(start, _M)], dstb.at[pl.ds(0, _M)])
    pltpu.sync_copy(l_hbm.at[pl.ds(start, _M)], lb)
    dstb[pl.ds(_M, 16)] = jnp.full((16,), -1, jnp.int32)

    def step_a(v, carry):
        csum, rbase = carry
        base = v * 16
        dv = dstb[pl.ds(base, 16)]
        dn = dstb[pl.ds(base + 1, 16)]
        ex = jnp.exp(lb[pl.ds(base, 16)])
        exb[pl.ds(base, 16)] = ex
        lastm = dv != dn
        lasti = lastm.astype(jnp.int32)
        c = plsc.cumsum(ex) + csum
        pcnt = plsc.cumsum(lasti)
        rid = rbase + pcnt - lasti
        ridb[pl.ds(base, 16)] = rid
        plsc.store_scatter(cb, [rid + 1], c, mask=lastm)
        return _take16(c, idx15), rbase + _take16(pcnt, idx15)

    csum0 = jnp.zeros((16,), jnp.float32)
    rb0 = jnp.zeros((16,), jnp.int32)
    _, rbF = lax.fori_loop(0, _M // 16, step_a, (csum0, rb0))
    num_runs = jnp.max(rbF)

    def step_b(r, _):
        base = r * 16
        hi = cb[pl.ds(base + 1, 16)]
        lo = cb[pl.ds(base, 16)]
        totb[pl.ds(base, 16)] = hi - lo
        return 0

    lax.fori_loop(0, (num_runs + 15) // 16, step_b, 0)

    d_first = _take16(dstb[pl.ds(0, 16)], zeros16)
    d_last = _take16(dstb[pl.ds(_M - 16, 16)], idx15)
    f_tot = plsc.load_gather(cb, [zeros16 + 1])
    last_idx = jnp.full((16,), num_runs - 1, jnp.int32)
    l_tot = plsc.load_gather(totb, [last_idx])
    multi = jnp.full((16,), num_runs, jnp.int32) > 1
    l_pub = jnp.where(multi, l_tot, jnp.zeros((16,), jnp.float32))
    pub = jnp.where(lane == 0, plsc.bitcast(d_first, jnp.float32),
          jnp.where(lane == 1, f_tot,
          jnp.where(lane == 2, plsc.bitcast(d_last, jnp.float32),
          jnp.where(lane == 3, l_pub, jnp.zeros((16,), jnp.float32)))))
    pubb[...] = pub
    pltpu.sync_copy(pubb.at[pl.ds(0, 8)], shtbl.at[pl.ds(wid * 8, 8)])
    plsc.subcore_barrier()
    pltpu.sync_copy(shtbl, tblb)

    lane8 = lane * 8
    fdst = plsc.bitcast(plsc.load_gather(tblb, [lane8]), jnp.int32)
    fsum = plsc.load_gather(tblb, [lane8 + 1])
    ldst = plsc.bitcast(plsc.load_gather(tblb, [lane8 + 2]), jnp.int32)
    lsum = plsc.load_gather(tblb, [lane8 + 3])

    def seg_total(d):
        zf = jnp.zeros((16,), jnp.float32)
        contrib = jnp.where(fdst == d, fsum, zf) + jnp.where(ldst == d, lsum, zf)
        return jnp.sum(contrib)

    cf = seg_total(d_first)
    cl = seg_total(d_last)
    lane0 = lane == 0
    plsc.store_scatter(totb, [zeros16], jnp.full((16,), cf), mask=lane0)
    plsc.store_scatter(totb, [last_idx], jnp.full((16,), cl), mask=lane0)

    def step_c(v, _):
        base = v * 16
        rid = ridb[pl.ds(base, 16)]
        tot = plsc.load_gather(totb, [rid])
        lb[pl.ds(base, 16)] = exb[pl.ds(base, 16)] / tot
        return 0

    lax.fori_loop(0, _M // 16, step_c, 0)
    pltpu.sync_copy(lb, w_hbm.at[pl.ds(start, _M)])


def _sc_softmax(l_flat, dst):
    mesh = plsc.VectorSubcoreMesh(
        core_axis_name="c", subcore_axis_name="s", num_cores=1
    )
    fn = functools.partial(
        pl.kernel,
        mesh=mesh,
        compiler_params=pltpu.CompilerParams(needs_layout_passes=False),
        out_type=jax.ShapeDtypeStruct((_E,), jnp.float32),
        scratch_types=[
            pltpu.VMEM((_M + 16,), jnp.int32),
            pltpu.VMEM((_M,), jnp.float32),
            pltpu.VMEM((_M,), jnp.float32),
            pltpu.VMEM((_M,), jnp.int32),
            pltpu.VMEM((_M + 16,), jnp.float32),
            pltpu.VMEM((_M + 16,), jnp.float32),
            pltpu.VMEM((16,), jnp.float32),
            pltpu.VMEM((128,), jnp.float32),
            pltpu.VMEM_SHARED((128,), jnp.float32),
        ],
    )(_sc_softmax_body)
    return fn(l_flat, dst)


def _pool_body(g_ref, w_ref, x_ref, o_ref):
    @pl.when(pl.program_id(0) == 0)
    def _init():
        o_ref[...] = jnp.zeros_like(o_ref)

    g = g_ref[...]
    onehot = (g == lax.broadcasted_iota(jnp.int32, (_BP, _G), 1)).astype(
        jnp.float32
    )
    wx = x_ref[...] * w_ref[...]
    o_ref[...] += lax.dot_general(
        onehot, wx, (((0,), (0,)), ((), ())), preferred_element_type=jnp.float32
    )


def _pool(graph_ids2d, w2d, edge_feats):
    return pl.pallas_call(
        _pool_body,
        grid=(_E // _BP,),
        in_specs=[
            pl.BlockSpec((_BP, 1), lambda i: (i, 0)),
            pl.BlockSpec((_BP, 1), lambda i: (i, 0)),
            pl.BlockSpec((_BP, _D), lambda i: (i, 0)),
        ],
        out_specs=pl.BlockSpec((_G, _D), lambda i: (0, 0)),
        out_shape=jax.ShapeDtypeStruct((_G, _D), jnp.float32),
    )(graph_ids2d, w2d, edge_feats)


def kernel(edge_feats, dst, graph_ids, W, b):
    dst = dst.astype(jnp.int32)
    graph_ids = graph_ids.astype(jnp.int32)
    logits = _logits(edge_feats, W, b)
    w_flat = _sc_softmax(logits.reshape(_E), dst)
    w2d = w_flat.reshape(_E, 1)
    h = _pool(graph_ids.reshape(_E, 1), w2d, edge_feats)
    return (h, w2d)


# speedup vs baseline: 12.4503x; 1.3760x over previous
import functools

import jax
import jax.numpy as jnp
from jax import lax
from jax.experimental import pallas as pl
from jax.experimental.pallas import tpu as pltpu
from jax.experimental.pallas import tpu_sc as plsc

_E = 320000
_D = 128
_G = 64
_NTILES = 16
_M = _E // _NTILES

_BE = 8000
_BP = 8000


def _logits_body(x_ref, w_ref, b_ref, o_ref):
    x = x_ref[...]
    z = jnp.dot(x, w_ref[...], preferred_element_type=jnp.float32)
    z = z + b_ref[0, 0]
    o_ref[...] = jnp.where(z >= 0, z, 0.01 * z)


def _logits(edge_feats, W, b):
    return pl.pallas_call(
        _logits_body,
        grid=(_E // _BE,),
        in_specs=[
            pl.BlockSpec((_BE, _D), lambda i: (i, 0)),
            pl.BlockSpec((_D, 1), lambda i: (0, 0)),
            pl.BlockSpec((1, 1), lambda i: (0, 0)),
        ],
        out_specs=pl.BlockSpec((_BE, 1), lambda i: (i, 0)),
        out_shape=jax.ShapeDtypeStruct((_E, 1), jnp.float32),
    )(edge_feats, W, b.reshape(1, 1))


def _take16(x, idx):
    return lax.gather(
        x,
        idx[:, None],
        lax.GatherDimensionNumbers(
            offset_dims=(), collapsed_slice_dims=(0,), start_index_map=(0,)
        ),
        (1,),
        mode=lax.GatherScatterMode.PROMISE_IN_BOUNDS,
    )


def _sc_softmax_body(l_hbm, dst_hbm, w_hbm, dstb, lb, exb, ridb, cb, totb,
                     pubb, tblb, shtbl):
    wid = lax.axis_index("s")
    start = wid * _M
    lane = lax.iota(jnp.int32, 16)
    zeros16 = jnp.zeros((16,), jnp.int32)
    idx15 = jnp.full((16,), 15, jnp.int32)

    pltpu.sync_copy(dst_hbm.at[pl.ds(start, _M)], dstb.at[pl.ds(0, _M)])
    pltpu.sync_copy(l_hbm.at[pl.ds(start, _M)], lb)
    dstb[pl.ds(_M, 16)] = jnp.full((16,), -1, jnp.int32)

    def step_a(v, carry):
        csum, rbase = carry
        base = v * 16
        dv = dstb[pl.ds(base, 16)]
        dn = dstb[pl.ds(base + 1, 16)]
        ex = jnp.exp(lb[pl.ds(base, 16)])
        exb[pl.ds(base, 16)] = ex
        lastm = dv != dn
        lasti = lastm.astype(jnp.int32)
        c = plsc.cumsum(ex) + csum
        pcnt = plsc.cumsum(lasti)
        rid = rbase + pcnt - lasti
        ridb[pl.ds(base, 16)] = rid
        plsc.store_scatter(cb, [rid + 1], c, mask=lastm)
        return _take16(c, idx15), rbase + _take16(pcnt, idx15)

    csum0 = jnp.zeros((16,), jnp.float32)
    rb0 = jnp.zeros((16,), jnp.int32)
    _, rbF = lax.fori_loop(0, _M // 16, step_a, (csum0, rb0))
    num_runs = jnp.max(rbF)

    def step_b(r, _):
        base = r * 16
        hi = cb[pl.ds(base + 1, 16)]
        lo = cb[pl.ds(base, 16)]
        totb[pl.ds(base, 16)] = hi - lo
        return 0

    lax.fori_loop(0, (num_runs + 15) // 16, step_b, 0)

    d_first = _take16(dstb[pl.ds(0, 16)], zeros16)
    d_last = _take16(dstb[pl.ds(_M - 16, 16)], idx15)
    f_tot = plsc.load_gather(cb, [zeros16 + 1])
    last_idx = jnp.full((16,), num_runs - 1, jnp.int32)
    l_tot = plsc.load_gather(totb, [last_idx])
    multi = jnp.full((16,), num_runs, jnp.int32) > 1
    l_pub = jnp.where(multi, l_tot, jnp.zeros((16,), jnp.float32))
    pub = jnp.where(lane == 0, plsc.bitcast(d_first, jnp.float32),
          jnp.where(lane == 1, f_tot,
          jnp.where(lane == 2, plsc.bitcast(d_last, jnp.float32),
          jnp.where(lane == 3, l_pub, jnp.zeros((16,), jnp.float32)))))
    pubb[...] = pub
    pltpu.sync_copy(pubb.at[pl.ds(0, 8)], shtbl.at[pl.ds(wid * 8, 8)])
    plsc.subcore_barrier()
    pltpu.sync_copy(shtbl, tblb)

    lane8 = lane * 8
    fdst = plsc.bitcast(plsc.load_gather(tblb, [lane8]), jnp.int32)
    fsum = plsc.load_gather(tblb, [lane8 + 1])
    ldst = plsc.bitcast(plsc.load_gather(tblb, [lane8 + 2]), jnp.int32)
    lsum = plsc.load_gather(tblb, [lane8 + 3])

    def seg_total(d):
        zf = jnp.zeros((16,), jnp.float32)
        contrib = jnp.where(fdst == d, fsum, zf) + jnp.where(ldst == d, lsum, zf)
        return jnp.sum(contrib)

    cf = seg_total(d_first)
    cl = seg_total(d_last)
    lane0 = lane == 0
    plsc.store_scatter(totb, [zeros16], jnp.full((16,), cf), mask=lane0)
    plsc.store_scatter(totb, [last_idx], jnp.full((16,), cl), mask=lane0)

    def step_c(v, _):
        base = v * 16
        rid = ridb[pl.ds(base, 16)]
        tot = plsc.load_gather(totb, [rid])
        lb[pl.ds(base, 16)] = exb[pl.ds(base, 16)] / tot
        return 0

    lax.fori_loop(0, _M // 16, step_c, 0)
    pltpu.sync_copy(lb, w_hbm.at[pl.ds(start, _M)])


def _sc_softmax(l_flat, dst):
    mesh = plsc.VectorSubcoreMesh(
        core_axis_name="c", subcore_axis_name="s", num_cores=1
    )
    fn = functools.partial(
        pl.kernel,
        mesh=mesh,
        compiler_params=pltpu.CompilerParams(needs_layout_passes=False),
        out_type=jax.ShapeDtypeStruct((_E,), jnp.float32),
        scratch_types=[
            pltpu.VMEM((_M + 16,), jnp.int32),
            pltpu.VMEM((_M,), jnp.float32),
            pltpu.VMEM((_M,), jnp.float32),
            pltpu.VMEM((_M,), jnp.int32),
            pltpu.VMEM((_M + 16,), jnp.float32),
            pltpu.VMEM((_M + 16,), jnp.float32),
            pltpu.VMEM((16,), jnp.float32),
            pltpu.VMEM((128,), jnp.float32),
            pltpu.VMEM_SHARED((128,), jnp.float32),
        ],
    )(_sc_softmax_body)
    return fn(l_flat, dst)


def _pool_body(g_ref, w_ref, x_ref, o_ref):
    @pl.when(pl.program_id(0) == 0)
    def _init():
        o_ref[...] = jnp.zeros_like(o_ref)

    g = g_ref[0]
    onehot_t = (g == lax.broadcasted_iota(jnp.int32, (_G, _BP), 0)).astype(
        jnp.float32
    )
    wx = x_ref[...] * w_ref[...]
    o_ref[...] += lax.dot_general(
        onehot_t, wx, (((1,), (0,)), ((), ())),
        preferred_element_type=jnp.float32,
    )


def _pool(graph_ids3d, w2d, edge_feats):
    return pl.pallas_call(
        _pool_body,
        grid=(_E // _BP,),
        in_specs=[
            pl.BlockSpec((1, 1, _BP), lambda i: (i, 0, 0)),
            pl.BlockSpec((_BP, 1), lambda i: (i, 0)),
            pl.BlockSpec((_BP, _D), lambda i: (i, 0)),
        ],
        out_specs=pl.BlockSpec((_G, _D), lambda i: (0, 0)),
        out_shape=jax.ShapeDtypeStruct((_G, _D), jnp.float32),
    )(graph_ids3d, w2d, edge_feats)


def kernel(edge_feats, dst, graph_ids, W, b):
    dst = dst.astype(jnp.int32)
    graph_ids = graph_ids.astype(jnp.int32)
    logits = _logits(edge_feats, W, b)
    w_flat = _sc_softmax(logits.reshape(_E), dst)
    w2d = w_flat.reshape(_E, 1)
    h = _pool(graph_ids.reshape(_E // _BP, 1, _BP), w2d, edge_feats)
    return (h, w2d)


# speedup vs baseline: 25.5019x; 2.0483x over previous
import functools

import jax
import jax.numpy as jnp
from jax import lax
from jax.experimental import pallas as pl
from jax.experimental.pallas import tpu as pltpu
from jax.experimental.pallas import tpu_sc as plsc

_E = 320000
_D = 128
_G = 64
_NTILES = 16
_M = _E // _NTILES

_BE = 8192
_BP = 8000


def _logits_body(x_ref, w_ref, b_ref, o_ref):
    x = x_ref[...]
    z2 = lax.dot_general(
        w_ref[...], x, (((0,), (1,)), ((), ())),
        preferred_element_type=jnp.float32,
    )
    z = z2[0] + b_ref[0, 0]
    o_ref[...] = jnp.where(z >= 0, z, 0.01 * z)


def _logits(edge_feats, W, b):
    return pl.pallas_call(
        _logits_body,
        grid=((_E + _BE - 1) // _BE,),
        in_specs=[
            pl.BlockSpec((_BE, _D), lambda i: (i, 0)),
            pl.BlockSpec((_D, 1), lambda i: (0, 0)),
            pl.BlockSpec((1, 1), lambda i: (0, 0)),
        ],
        out_specs=pl.BlockSpec((_BE,), lambda i: (i,)),
        out_shape=jax.ShapeDtypeStruct((_E,), jnp.float32),
    )(edge_feats, W, b.reshape(1, 1))


def _take16(x, idx):
    return lax.gather(
        x,
        idx[:, None],
        lax.GatherDimensionNumbers(
            offset_dims=(), collapsed_slice_dims=(0,), start_index_map=(0,)
        ),
        (1,),
        mode=lax.GatherScatterMode.PROMISE_IN_BOUNDS,
    )


def _sc_softmax_body(l_hbm, dst_hbm, w_hbm, dstb, lb, exb, ridb, cb, totb,
                     pubb, tblb, shtbl):
    wid = lax.axis_index("s")
    start = wid * _M
    lane = lax.iota(jnp.int32, 16)
    zeros16 = jnp.zeros((16,), jnp.int32)
    idx15 = jnp.full((16,), 15, jnp.int32)

    pltpu.sync_copy(dst_hbm.at[pl.ds(start, _M)], dstb.at[pl.ds(0, _M)])
    pltpu.sync_copy(l_hbm.at[pl.ds(start, _M)], lb)
    dstb[pl.ds(_M, 16)] = jnp.full((16,), -1, jnp.int32)

    def step_a(v, carry):
        csum, rbase = carry
        base = v * 16
        dv = dstb[pl.ds(base, 16)]
        dn = dstb[pl.ds(base + 1, 16)]
        ex = jnp.exp(lb[pl.ds(base, 16)])
        exb[pl.ds(base, 16)] = ex
        lastm = dv != dn
        lasti = lastm.astype(jnp.int32)
        c = plsc.cumsum(ex) + csum
        pcnt = plsc.cumsum(lasti)
        rid = rbase + pcnt - lasti
        ridb[pl.ds(base, 16)] = rid
        plsc.store_scatter(cb, [rid + 1], c, mask=lastm)
        return _take16(c, idx15), rbase + _take16(pcnt, idx15)

    csum0 = jnp.zeros((16,), jnp.float32)
    rb0 = jnp.zeros((16,), jnp.int32)
    _, rbF = lax.fori_loop(0, _M // 16, step_a, (csum0, rb0))
    num_runs = jnp.max(rbF)

    def step_b(r, _):
        base = r * 16
        hi = cb[pl.ds(base + 1, 16)]
        lo = cb[pl.ds(base, 16)]
        totb[pl.ds(base, 16)] = hi - lo
        return 0

    lax.fori_loop(0, (num_runs + 15) // 16, step_b, 0)

    d_first = _take16(dstb[pl.ds(0, 16)], zeros16)
    d_last = _take16(dstb[pl.ds(_M - 16, 16)], idx15)
    f_tot = plsc.load_gather(cb, [zeros16 + 1])
    last_idx = jnp.full((16,), num_runs - 1, jnp.int32)
    l_tot = plsc.load_gather(totb, [last_idx])
    multi = jnp.full((16,), num_runs, jnp.int32) > 1
    l_pub = jnp.where(multi, l_tot, jnp.zeros((16,), jnp.float32))
    pub = jnp.where(lane == 0, plsc.bitcast(d_first, jnp.float32),
          jnp.where(lane == 1, f_tot,
          jnp.where(lane == 2, plsc.bitcast(d_last, jnp.float32),
          jnp.where(lane == 3, l_pub, jnp.zeros((16,), jnp.float32)))))
    pubb[...] = pub
    pltpu.sync_copy(pubb.at[pl.ds(0, 8)], shtbl.at[pl.ds(wid * 8, 8)])
    plsc.subcore_barrier()
    pltpu.sync_copy(shtbl, tblb)

    lane8 = lane * 8
    fdst = plsc.bitcast(plsc.load_gather(tblb, [lane8]), jnp.int32)
    fsum = plsc.load_gather(tblb, [lane8 + 1])
    ldst = plsc.bitcast(plsc.load_gather(tblb, [lane8 + 2]), jnp.int32)
    lsum = plsc.load_gather(tblb, [lane8 + 3])

    def seg_total(d):
        zf = jnp.zeros((16,), jnp.float32)
        contrib = jnp.where(fdst == d, fsum, zf) + jnp.where(ldst == d, lsum, zf)
        return jnp.sum(contrib)

    cf = seg_total(d_first)
    cl = seg_total(d_last)
    lane0 = lane == 0
    plsc.store_scatter(totb, [zeros16], jnp.full((16,), cf), mask=lane0)
    plsc.store_scatter(totb, [last_idx], jnp.full((16,), cl), mask=lane0)

    def step_c(v, _):
        base = v * 16
        rid = ridb[pl.ds(base, 16)]
        tot = plsc.load_gather(totb, [rid])
        lb[pl.ds(base, 16)] = exb[pl.ds(base, 16)] / tot
        return 0

    lax.fori_loop(0, _M // 16, step_c, 0)
    pltpu.sync_copy(lb, w_hbm.at[pl.ds(start, _M)])


def _sc_softmax(l_flat, dst):
    mesh = plsc.VectorSubcoreMesh(
        core_axis_name="c", subcore_axis_name="s", num_cores=1
    )
    fn = functools.partial(
        pl.kernel,
        mesh=mesh,
        compiler_params=pltpu.CompilerParams(needs_layout_passes=False),
        out_type=jax.ShapeDtypeStruct((_E,), jnp.float32),
        scratch_types=[
            pltpu.VMEM((_M + 16,), jnp.int32),
            pltpu.VMEM((_M,), jnp.float32),
            pltpu.VMEM((_M,), jnp.float32),
            pltpu.VMEM((_M,), jnp.int32),
            pltpu.VMEM((_M + 16,), jnp.float32),
            pltpu.VMEM((_M + 16,), jnp.float32),
            pltpu.VMEM((16,), jnp.float32),
            pltpu.VMEM((128,), jnp.float32),
            pltpu.VMEM_SHARED((128,), jnp.float32),
        ],
    )(_sc_softmax_body)
    return fn(l_flat, dst)


def _pool_body(g_ref, w_ref, x_ref, o_ref):
    @pl.when(pl.program_id(0) == 0)
    def _init():
        o_ref[...] = jnp.zeros_like(o_ref)

    g = g_ref[0]
    w = w_ref[0]
    onehot_t = (g == lax.broadcasted_iota(jnp.int32, (_G, _BP), 0)).astype(
        jnp.float32
    )
    w_onehot = onehot_t * w
    o_ref[...] += lax.dot_general(
        w_onehot, x_ref[...], (((1,), (0,)), ((), ())),
        preferred_element_type=jnp.float32,
    )


def _pool(graph_ids3d, w3d, edge_feats):
    return pl.pallas_call(
        _pool_body,
        grid=(_E // _BP,),
        in_specs=[
            pl.BlockSpec((1, 1, _BP), lambda i: (i, 0, 0)),
            pl.BlockSpec((1, 1, _BP), lambda i: (i, 0, 0)),
            pl.BlockSpec((_BP, _D), lambda i: (i, 0)),
        ],
        out_specs=pl.BlockSpec((_G, _D), lambda i: (0, 0)),
        out_shape=jax.ShapeDtypeStruct((_G, _D), jnp.float32),
    )(graph_ids3d, w3d, edge_feats)


def kernel(edge_feats, dst, graph_ids, W, b):
    dst = dst.astype(jnp.int32)
    graph_ids = graph_ids.astype(jnp.int32)
    logits = _logits(edge_feats, W, b)
    w_flat = _sc_softmax(logits, dst)
    nb = _E // _BP
    h = _pool(graph_ids.reshape(nb, 1, _BP), w_flat.reshape(nb, 1, _BP),
              edge_feats)
    return (h, w_flat.reshape(_E, 1))


# speedup vs baseline: 25.7539x; 1.0099x over previous
import functools

import jax
import jax.numpy as jnp
from jax import lax
from jax.experimental import pallas as pl
from jax.experimental.pallas import tpu as pltpu
from jax.experimental.pallas import tpu_sc as plsc

_E = 320000
_D = 128
_G = 64
_NTILES = 16
_M = _E // _NTILES
_UNROLL = 5

_BE = 8192
_BP = 8000


def _logits_body(x_ref, w_ref, b_ref, o_ref):
    x = x_ref[...]
    z2 = lax.dot_general(
        w_ref[...], x, (((0,), (1,)), ((), ())),
        preferred_element_type=jnp.float32,
    )
    z = z2[0] + b_ref[0, 0]
    o_ref[...] = jnp.where(z >= 0, z, 0.01 * z)


def _logits(edge_feats, W, b):
    return pl.pallas_call(
        _logits_body,
        grid=((_E + _BE - 1) // _BE,),
        in_specs=[
            pl.BlockSpec((_BE, _D), lambda i: (i, 0)),
            pl.BlockSpec((_D, 1), lambda i: (0, 0)),
            pl.BlockSpec((1, 1), lambda i: (0, 0)),
        ],
        out_specs=pl.BlockSpec((_BE,), lambda i: (i,)),
        out_shape=jax.ShapeDtypeStruct((_E,), jnp.float32),
    )(edge_feats, W, b.reshape(1, 1))


def _take16(x, idx):
    return lax.gather(
        x,
        idx[:, None],
        lax.GatherDimensionNumbers(
            offset_dims=(), collapsed_slice_dims=(0,), start_index_map=(0,)
        ),
        (1,),
        mode=lax.GatherScatterMode.PROMISE_IN_BOUNDS,
    )


def _sc_softmax_body(l_hbm, dst_hbm, w_hbm, dstb, lb, exb, ridb, cb, totb,
                     pubb, tblb, shtbl):
    wid = lax.axis_index("s")
    start = wid * _M
    lane = lax.iota(jnp.int32, 16)
    zeros16 = jnp.zeros((16,), jnp.int32)
    idx15 = jnp.full((16,), 15, jnp.int32)

    pltpu.sync_copy(dst_hbm.at[pl.ds(start, _M)], dstb.at[pl.ds(0, _M)])
    pltpu.sync_copy(l_hbm.at[pl.ds(start, _M)], lb)
    dstb[pl.ds(_M, 16)] = jnp.full((16,), -1, jnp.int32)

    def step_a(v, carry):
        csum, rbase = carry
        for k in range(_UNROLL):
            base = (v * _UNROLL + k) * 16
            dv = dstb[pl.ds(base, 16)]
            dn = dstb[pl.ds(base + 1, 16)]
            ex = jnp.exp(lb[pl.ds(base, 16)])
            exb[pl.ds(base, 16)] = ex
            lastm = dv != dn
            lasti = lastm.astype(jnp.int32)
            c = plsc.cumsum(ex) + csum
            pcnt = plsc.cumsum(lasti)
            rid = rbase + pcnt - lasti
            ridb[pl.ds(base, 16)] = rid
            plsc.store_scatter(cb, [rid + 1], c, mask=lastm)
            csum = _take16(c, idx15)
            rbase = rbase + _take16(pcnt, idx15)
        return csum, rbase

    csum0 = jnp.zeros((16,), jnp.float32)
    rb0 = jnp.zeros((16,), jnp.int32)
    _, rbF = lax.fori_loop(0, _M // (16 * _UNROLL), step_a, (csum0, rb0))
    num_runs = jnp.max(rbF)

    def step_b(r, _):
        base = r * 16
        hi = cb[pl.ds(base + 1, 16)]
        lo = cb[pl.ds(base, 16)]
        totb[pl.ds(base, 16)] = hi - lo
        return 0

    lax.fori_loop(0, (num_runs + 15) // 16, step_b, 0)

    d_first = _take16(dstb[pl.ds(0, 16)], zeros16)
    d_last = _take16(dstb[pl.ds(_M - 16, 16)], idx15)
    f_tot = plsc.load_gather(cb, [zeros16 + 1])
    last_idx = jnp.full((16,), num_runs - 1, jnp.int32)
    l_tot = plsc.load_gather(totb, [last_idx])
    multi = jnp.full((16,), num_runs, jnp.int32) > 1
    l_pub = jnp.where(multi, l_tot, jnp.zeros((16,), jnp.float32))
    pub = jnp.where(lane == 0, plsc.bitcast(d_first, jnp.float32),
          jnp.where(lane == 1, f_tot,
          jnp.where(lane == 2, plsc.bitcast(d_last, jnp.float32),
          jnp.where(lane == 3, l_pub, jnp.zeros((16,), jnp.float32)))))
    pubb[...] = pub
    pltpu.sync_copy(pubb.at[pl.ds(0, 8)], shtbl.at[pl.ds(wid * 8, 8)])
    plsc.subcore_barrier()
    pltpu.sync_copy(shtbl, tblb)

    lane8 = lane * 8
    fdst = plsc.bitcast(plsc.load_gather(tblb, [lane8]), jnp.int32)
    fsum = plsc.load_gather(tblb, [lane8 + 1])
    ldst = plsc.bitcast(plsc.load_gather(tblb, [lane8 + 2]), jnp.int32)
    lsum = plsc.load_gather(tblb, [lane8 + 3])

    def seg_total(d):
        zf = jnp.zeros((16,), jnp.float32)
        contrib = jnp.where(fdst == d, fsum, zf) + jnp.where(ldst == d, lsum, zf)
        return jnp.sum(contrib)

    cf = seg_total(d_first)
    cl = seg_total(d_last)
    lane0 = lane == 0
    plsc.store_scatter(totb, [zeros16], jnp.full((16,), cf), mask=lane0)
    plsc.store_scatter(totb, [last_idx], jnp.full((16,), cl), mask=lane0)

    def step_c(v, _):
        for k in range(_UNROLL):
            base = (v * _UNROLL + k) * 16
            rid = ridb[pl.ds(base, 16)]
            tot = plsc.load_gather(totb, [rid])
            lb[pl.ds(base, 16)] = exb[pl.ds(base, 16)] / tot
        return 0

    lax.fori_loop(0, _M // (16 * _UNROLL), step_c, 0)
    pltpu.sync_copy(lb, w_hbm.at[pl.ds(start, _M)])


def _sc_softmax(l_flat, dst):
    mesh = plsc.VectorSubcoreMesh(
        core_axis_name="c", subcore_axis_name="s", num_cores=1
    )
    fn = functools.partial(
        pl.kernel,
        mesh=mesh,
        compiler_params=pltpu.CompilerParams(needs_layout_passes=False),
        out_type=jax.ShapeDtypeStruct((_E,), jnp.float32),
        scratch_types=[
            pltpu.VMEM((_M + 16,), jnp.int32),
            pltpu.VMEM((_M,), jnp.float32),
            pltpu.VMEM((_M,), jnp.float32),
            pltpu.VMEM((_M,), jnp.int32),
            pltpu.VMEM((_M + 16,), jnp.float32),
            pltpu.VMEM((_M + 16,), jnp.float32),
            pltpu.VMEM((16,), jnp.float32),
            pltpu.VMEM((128,), jnp.float32),
            pltpu.VMEM_SHARED((128,), jnp.float32),
        ],
    )(_sc_softmax_body)
    return fn(l_flat, dst)


def _pool_body(g_ref, w_ref, x_ref, o_ref):
    @pl.when(pl.program_id(0) == 0)
    def _init():
        o_ref[...] = jnp.zeros_like(o_ref)

    g = g_ref[0]
    w = w_ref[0]
    onehot_t = (g == lax.broadcasted_iota(jnp.int32, (_G, _BP), 0)).astype(
        jnp.float32
    )
    w_onehot = onehot_t * w
    o_ref[...] += lax.dot_general(
        w_onehot, x_ref[...], (((1,), (0,)), ((), ())),
        preferred_element_type=jnp.float32,
    )


def _pool(graph_ids3d, w3d, edge_feats):
    return pl.pallas_call(
        _pool_body,
        grid=(_E // _BP,),
        in_specs=[
            pl.BlockSpec((1, 1, _BP), lambda i: (i, 0, 0)),
            pl.BlockSpec((1, 1, _BP), lambda i: (i, 0, 0)),
            pl.BlockSpec((_BP, _D), lambda i: (i, 0)),
        ],
        out_specs=pl.BlockSpec((_G, _D), lambda i: (0, 0)),
        out_shape=jax.ShapeDtypeStruct((_G, _D), jnp.float32),
    )(graph_ids3d, w3d, edge_feats)


def kernel(edge_feats, dst, graph_ids, W, b):
    dst = dst.astype(jnp.int32)
    graph_ids = graph_ids.astype(jnp.int32)
    logits = _logits(edge_feats, W, b)
    w_flat = _sc_softmax(logits, dst)
    nb = _E // _BP
    h = _pool(graph_ids.reshape(nb, 1, _BP), w_flat.reshape(nb, 1, _BP),
              edge_feats)
    return (h, w_flat.reshape(_E, 1))


# speedup vs baseline: 28.2682x; 1.0976x over previous
import functools

import jax
import jax.numpy as jnp
from jax import lax
from jax.experimental import pallas as pl
from jax.experimental.pallas import tpu as pltpu
from jax.experimental.pallas import tpu_sc as plsc

_E = 320000
_D = 128
_G = 64
_NTILES = 16
_M = _E // _NTILES
_UNROLL = 5

_BE = 32768
_BP = 16000


def _logits_body(x_ref, w_ref, b_ref, o_ref):
    x = x_ref[...]
    z2 = lax.dot_general(
        w_ref[...], x, (((0,), (1,)), ((), ())),
        preferred_element_type=jnp.float32,
    )
    z = z2[0] + b_ref[0, 0]
    o_ref[...] = jnp.where(z >= 0, z, 0.01 * z)


def _logits(edge_feats, W, b):
    return pl.pallas_call(
        _logits_body,
        grid=((_E + _BE - 1) // _BE,),
        in_specs=[
            pl.BlockSpec((_BE, _D), lambda i: (i, 0)),
            pl.BlockSpec((_D, 1), lambda i: (0, 0)),
            pl.BlockSpec((1, 1), lambda i: (0, 0)),
        ],
        out_specs=pl.BlockSpec((_BE,), lambda i: (i,)),
        out_shape=jax.ShapeDtypeStruct((_E,), jnp.float32),
    )(edge_feats, W, b.reshape(1, 1))


def _take16(x, idx):
    return lax.gather(
        x,
        idx[:, None],
        lax.GatherDimensionNumbers(
            offset_dims=(), collapsed_slice_dims=(0,), start_index_map=(0,)
        ),
        (1,),
        mode=lax.GatherScatterMode.PROMISE_IN_BOUNDS,
    )


def _sc_softmax_body(l_hbm, dst_hbm, w_hbm, dstb, lb, exb, ridb, cb, totb,
                     pubb, tblb, shtbl):
    wid = lax.axis_index("s")
    start = wid * _M
    lane = lax.iota(jnp.int32, 16)
    zeros16 = jnp.zeros((16,), jnp.int32)
    idx15 = jnp.full((16,), 15, jnp.int32)

    pltpu.sync_copy(dst_hbm.at[pl.ds(start, _M)], dstb.at[pl.ds(0, _M)])
    pltpu.sync_copy(l_hbm.at[pl.ds(start, _M)], lb)
    dstb[pl.ds(_M, 16)] = jnp.full((16,), -1, jnp.int32)

    def step_a(v, carry):
        csum, rbase = carry
        for k in range(_UNROLL):
            base = (v * _UNROLL + k) * 16
            dv = dstb[pl.ds(base, 16)]
            dn = dstb[pl.ds(base + 1, 16)]
            ex = jnp.exp(lb[pl.ds(base, 16)])
            exb[pl.ds(base, 16)] = ex
            lastm = dv != dn
            lasti = lastm.astype(jnp.int32)
            c = plsc.cumsum(ex) + csum
            pcnt = plsc.cumsum(lasti)
            rid = rbase + pcnt - lasti
            ridb[pl.ds(base, 16)] = rid
            plsc.store_scatter(cb, [rid + 1], c, mask=lastm)
            csum = _take16(c, idx15)
            rbase = rbase + _take16(pcnt, idx15)
        return csum, rbase

    csum0 = jnp.zeros((16,), jnp.float32)
    rb0 = jnp.zeros((16,), jnp.int32)
    _, rbF = lax.fori_loop(0, _M // (16 * _UNROLL), step_a, (csum0, rb0))
    num_runs = jnp.max(rbF)

    def step_b(r, _):
        base = r * 16
        hi = cb[pl.ds(base + 1, 16)]
        lo = cb[pl.ds(base, 16)]
        totb[pl.ds(base, 16)] = hi - lo
        return 0

    lax.fori_loop(0, (num_runs + 15) // 16, step_b, 0)

    d_first = _take16(dstb[pl.ds(0, 16)], zeros16)
    d_last = _take16(dstb[pl.ds(_M - 16, 16)], idx15)
    f_tot = plsc.load_gather(cb, [zeros16 + 1])
    last_idx = jnp.full((16,), num_runs - 1, jnp.int32)
    l_tot = plsc.load_gather(totb, [last_idx])
    multi = jnp.full((16,), num_runs, jnp.int32) > 1
    l_pub = jnp.where(multi, l_tot, jnp.zeros((16,), jnp.float32))
    pub = jnp.where(lane == 0, plsc.bitcast(d_first, jnp.float32),
          jnp.where(lane == 1, f_tot,
          jnp.where(lane == 2, plsc.bitcast(d_last, jnp.float32),
          jnp.where(lane == 3, l_pub, jnp.zeros((16,), jnp.float32)))))
    pubb[...] = pub
    pltpu.sync_copy(pubb.at[pl.ds(0, 8)], shtbl.at[pl.ds(wid * 8, 8)])
    plsc.subcore_barrier()
    pltpu.sync_copy(shtbl, tblb)

    lane8 = lane * 8
    fdst = plsc.bitcast(plsc.load_gather(tblb, [lane8]), jnp.int32)
    fsum = plsc.load_gather(tblb, [lane8 + 1])
    ldst = plsc.bitcast(plsc.load_gather(tblb, [lane8 + 2]), jnp.int32)
    lsum = plsc.load_gather(tblb, [lane8 + 3])

    def seg_total(d):
        zf = jnp.zeros((16,), jnp.float32)
        contrib = jnp.where(fdst == d, fsum, zf) + jnp.where(ldst == d, lsum, zf)
        return jnp.sum(contrib)

    cf = seg_total(d_first)
    cl = seg_total(d_last)
    lane0 = lane == 0
    plsc.store_scatter(totb, [zeros16], jnp.full((16,), cf), mask=lane0)
    plsc.store_scatter(totb, [last_idx], jnp.full((16,), cl), mask=lane0)

    def step_c(v, _):
        for k in range(_UNROLL):
            base = (v * _UNROLL + k) * 16
            rid = ridb[pl.ds(base, 16)]
            tot = plsc.load_gather(totb, [rid])
            lb[pl.ds(base, 16)] = exb[pl.ds(base, 16)] / tot
        return 0

    lax.fori_loop(0, _M // (16 * _UNROLL), step_c, 0)
    pltpu.sync_copy(lb, w_hbm.at[pl.ds(start, _M)])


def _sc_softmax(l_flat, dst):
    mesh = plsc.VectorSubcoreMesh(
        core_axis_name="c", subcore_axis_name="s", num_cores=1
    )
    fn = functools.partial(
        pl.kernel,
        mesh=mesh,
        compiler_params=pltpu.CompilerParams(needs_layout_passes=False),
        out_type=jax.ShapeDtypeStruct((_E,), jnp.float32),
        scratch_types=[
            pltpu.VMEM((_M + 16,), jnp.int32),
            pltpu.VMEM((_M,), jnp.float32),
            pltpu.VMEM((_M,), jnp.float32),
            pltpu.VMEM((_M,), jnp.int32),
            pltpu.VMEM((_M + 16,), jnp.float32),
            pltpu.VMEM((_M + 16,), jnp.float32),
            pltpu.VMEM((16,), jnp.float32),
            pltpu.VMEM((128,), jnp.float32),
            pltpu.VMEM_SHARED((128,), jnp.float32),
        ],
    )(_sc_softmax_body)
    return fn(l_flat, dst)


def _pool_body(g_ref, w_ref, x_ref, o_ref):
    @pl.when(pl.program_id(0) == 0)
    def _init():
        o_ref[...] = jnp.zeros_like(o_ref)

    g = g_ref[0]
    w = w_ref[0]
    onehot_t = (g == lax.broadcasted_iota(jnp.int32, (_G, _BP), 0)).astype(
        jnp.float32
    )
    w_onehot = onehot_t * w
    o_ref[...] += lax.dot_general(
        w_onehot, x_ref[...], (((1,), (0,)), ((), ())),
        preferred_element_type=jnp.float32,
    )


def _pool(graph_ids3d, w3d, edge_feats):
    return pl.pallas_call(
        _pool_body,
        grid=(_E // _BP,),
        in_specs=[
            pl.BlockSpec((1, 1, _BP), lambda i: (i, 0, 0)),
            pl.BlockSpec((1, 1, _BP), lambda i: (i, 0, 0)),
            pl.BlockSpec((_BP, _D), lambda i: (i, 0)),
        ],
        out_specs=pl.BlockSpec((_G, _D), lambda i: (0, 0)),
        out_shape=jax.ShapeDtypeStruct((_G, _D), jnp.float32),
    )(graph_ids3d, w3d, edge_feats)


def kernel(edge_feats, dst, graph_ids, W, b):
    dst = dst.astype(jnp.int32)
    graph_ids = graph_ids.astype(jnp.int32)
    logits = _logits(edge_feats, W, b)
    w_flat = _sc_softmax(logits, dst)
    nb = _E // _BP
    h = _pool(graph_ids.reshape(nb, 1, _BP), w_flat.reshape(nb, 1, _BP),
              edge_feats)
    return (h, w_flat.reshape(_E, 1))
